# baseline (speedup 1.0000x reference)
import functools

import jax
import jax.numpy as jnp
from jax.experimental import pallas as pl

B, N = 64, 24
NF, EF = 64, 4
HID, MSG, PASSES = 64, 32, 3

BB = 4


def _mpnn_kernel(nodes_ref, eflat_ref, W_h_ref, W_e_ref, b_msg_ref,
                 W_u_ref, W_m_ref, b_u_ref, W_gh_ref, W_gx_ref,
                 W_oh_ref, W_ox_ref, out_ref):
    f32 = jnp.float32
    NN = N * N
    r_i = jax.lax.broadcasted_iota(jnp.int32, (N, NN), 1)
    n_i = jax.lax.broadcasted_iota(jnp.int32, (N, NN), 0)
    S0 = (r_i // N == n_i).astype(f32)
    r2_i = jax.lax.broadcasted_iota(jnp.int32, (NN, N), 0)
    g_i = jax.lax.broadcasted_iota(jnp.int32, (NN, N), 1)
    TileG = (r2_i % N == g_i).astype(f32)

    W_h = W_h_ref[...]
    W_e = W_e_ref[...]
    b_msg = b_msg_ref[...]
    W_u = W_u_ref[...]
    W_m = W_m_ref[...]
    b_u = b_u_ref[...]
    W_gh = W_gh_ref[...]
    W_gx = W_gx_ref[...]
    W_oh = W_oh_ref[...]
    W_ox = W_ox_ref[...]

    dot = functools.partial(jnp.dot, preferred_element_type=f32)

    for bb in range(BB):
        x = nodes_ref[bb]
        e2 = eflat_ref[bb]
        adj = jnp.sum(e2, axis=1, keepdims=True)
        maskf = (adj != 0.0).astype(f32)
        act_sum = dot(S0, adj)
        active = act_sum != 0.0
        E = dot(e2, W_e) + b_msg
        h = x
        for _ in range(PASSES):
            Hm = dot(h, W_h)
            T = jnp.tanh(E + dot(TileG, Hm))
            msg = dot(S0, T * maskf)
            upd = jnp.tanh(dot(h, W_u) + dot(msg, W_m) + b_u)
            h = jnp.where(active, upd, h)
        gate = jax.nn.sigmoid(dot(h, W_gh) + dot(x, W_gx))
        o = dot(h, W_oh) + dot(x, W_ox)
        gated = gate * o * active.astype(f32)
        out_ref[0, pl.ds(bb, 1), :] = jnp.sum(gated, axis=0, keepdims=True)


def kernel(nodes, edges, W_msg, b_msg, W_u, W_m, b_u, W_g, W_o):
    f32 = jnp.float32
    eflat = edges.reshape(B, N * N, EF)
    W_h = W_msg[:HID]
    W_e = W_msg[HID:]
    b_msg2 = b_msg.reshape(1, MSG)
    b_u2 = b_u.reshape(1, HID)
    W_gh, W_gx = W_g[:HID], W_g[HID:]
    W_oh, W_ox = W_o[:HID], W_o[HID:]

    full = lambda shape: pl.BlockSpec(shape, lambda i: (0,) * len(shape))
    out = pl.pallas_call(
        _mpnn_kernel,
        grid=(B // BB,),
        in_specs=[
            pl.BlockSpec((BB, N, NF), lambda i: (i, 0, 0)),
            pl.BlockSpec((BB, N * N, EF), lambda i: (i, 0, 0)),
            full((HID, MSG)),
            full((EF, MSG)),
            full((1, MSG)),
            full((HID, HID)),
            full((MSG, HID)),
            full((1, HID)),
            full((HID, HID)),
            full((NF, HID)),
            full((HID, HID)),
            full((NF, HID)),
        ],
        out_specs=pl.BlockSpec((1, BB, HID), lambda i: (i, 0, 0)),
        out_shape=jax.ShapeDtypeStruct((B // BB, BB, HID), f32),
    )(nodes, eflat, W_h, W_e, b_msg2, W_u, W_m, b_u2,
      W_gh, W_gx, W_oh, W_ox)
    return out.reshape(B, HID)


# speedup vs baseline: 2.0362x; 2.0362x over previous
import functools

import jax
import jax.numpy as jnp
from jax.experimental import pallas as pl

B, N = 64, 24
NF, EF = 64, 4
HID, MSG, PASSES = 64, 32, 3

Q = 4
G = B // Q
GB = 2


def _mpnn_kernel(x_ref, e_ref, Wk_h_ref, Wk_e_ref, A32_ref, A64_ref,
                 b_msg_ref, Wk_u_ref, Wk_m_ref, b_u_ref, Wk_gh_ref,
                 Wk_gx_ref, Wk_oh_ref, Wk_ox_ref, out_ref):
    f32 = jnp.float32
    NN = N * N
    r_i = jax.lax.broadcasted_iota(jnp.int32, (N, NN), 1)
    n_i = jax.lax.broadcasted_iota(jnp.int32, (N, NN), 0)
    S0 = (r_i // N == n_i).astype(f32)
    r2_i = jax.lax.broadcasted_iota(jnp.int32, (NN, N), 0)
    g_i = jax.lax.broadcasted_iota(jnp.int32, (NN, N), 1)
    TileG = (r2_i % N == g_i).astype(f32)

    Wk_h = Wk_h_ref[...]
    Wk_e = Wk_e_ref[...]
    A32 = A32_ref[...]
    A64 = A64_ref[...]
    b_msg = b_msg_ref[...]
    Wk_u = Wk_u_ref[...]
    Wk_m = Wk_m_ref[...]
    b_u = b_u_ref[...]
    Wk_gh = Wk_gh_ref[...]
    Wk_gx = Wk_gx_ref[...]
    Wk_oh = Wk_oh_ref[...]
    Wk_ox = Wk_ox_ref[...]

    dot = functools.partial(jnp.dot, preferred_element_type=f32)

    for gb in range(GB):
        x4 = x_ref[gb]
        e4 = e_ref[gb]
        E4 = dot(e4, Wk_e) + b_msg
        adj32 = dot(e4, A32)
        mask4 = (adj32 != 0.0).astype(f32)
        act64 = dot(dot(S0, e4), A64) != 0.0
        h4 = x4
        for _ in range(PASSES):
            Hm4 = dot(h4, Wk_h)
            T4 = jnp.tanh(E4 + dot(TileG, Hm4))
            msg4 = dot(S0, T4 * mask4)
            upd4 = jnp.tanh(dot(h4, Wk_u) + dot(msg4, Wk_m) + b_u)
            h4 = jnp.where(act64, upd4, h4)
        gate4 = jax.nn.sigmoid(dot(h4, Wk_gh) + dot(x4, Wk_gx))
        o4 = dot(h4, Wk_oh) + dot(x4, Wk_ox)
        gated = gate4 * o4 * act64.astype(f32)
        out_ref[gb] = jnp.sum(gated, axis=0, keepdims=True)


def kernel(nodes, edges, W_msg, b_msg, W_u, W_m, b_u, W_g, W_o):
    f32 = jnp.float32
    eye = jnp.eye(Q, dtype=f32)
    kron = lambda w: jnp.kron(eye, w)
    e4 = edges.reshape(G, Q, N * N, EF).transpose(0, 2, 1, 3) \
              .reshape(G, N * N, Q * EF)
    x4 = nodes.reshape(G, Q, N, NF).transpose(0, 2, 1, 3) \
              .reshape(G, N, Q * NF)
    Wk_h = kron(W_msg[:HID])
    Wk_e = kron(W_msg[HID:])
    A32 = kron(jnp.ones((EF, MSG), f32))
    A64 = kron(jnp.ones((EF, HID), f32))
    b_msg4 = jnp.tile(b_msg, Q).reshape(1, Q * MSG)
    Wk_u = kron(W_u)
    Wk_m = kron(W_m)
    b_u4 = jnp.tile(b_u, Q).reshape(1, Q * HID)
    Wk_gh, Wk_gx = kron(W_g[:HID]), kron(W_g[HID:])
    Wk_oh, Wk_ox = kron(W_o[:HID]), kron(W_o[HID:])

    full = lambda shape: pl.BlockSpec(shape, lambda i: (0,) * len(shape))
    out = pl.pallas_call(
        _mpnn_kernel,
        grid=(G // GB,),
        in_specs=[
            pl.BlockSpec((GB, N, Q * NF), lambda i: (i, 0, 0)),
            pl.BlockSpec((GB, N * N, Q * EF), lambda i: (i, 0, 0)),
            full((Q * HID, Q * MSG)),
            full((Q * EF, Q * MSG)),
            full((Q * EF, Q * MSG)),
            full((Q * EF, Q * HID)),
            full((1, Q * MSG)),
            full((Q * HID, Q * HID)),
            full((Q * MSG, Q * HID)),
            full((1, Q * HID)),
            full((Q * HID, Q * HID)),
            full((Q * NF, Q * HID)),
            full((Q * HID, Q * HID)),
            full((Q * NF, Q * HID)),
        ],
        out_specs=pl.BlockSpec((GB, 1, Q * HID), lambda i: (i, 0, 0)),
        out_shape=jax.ShapeDtypeStruct((G, 1, Q * HID), f32),
    )(x4, e4, Wk_h, Wk_e, A32, A64, b_msg4, Wk_u, Wk_m, b_u4,
      Wk_gh, Wk_gx, Wk_oh, Wk_ox)
    return out.reshape(B, HID)


# speedup vs baseline: 2.2519x; 1.1059x over previous
import functools

import jax
import jax.numpy as jnp
from jax.experimental import pallas as pl

B, N = 64, 24
NF, EF = 64, 4
HID, MSG, PASSES = 64, 32, 3

Q = 4
G = B // Q
GB = 2


def _blockdiag_mask(rows, cols, rblk, cblk):
    r = jax.lax.broadcasted_iota(jnp.int32, (rows, cols), 0)
    c = jax.lax.broadcasted_iota(jnp.int32, (rows, cols), 1)
    return (r // rblk == c // cblk).astype(jnp.float32)


def _mpnn_kernel(x_ref, e_ref, W_msg_ref, b_msg_ref, W_u_ref, W_m_ref,
                 b_u_ref, W_g_ref, W_o_ref, out_ref):
    f32 = jnp.float32
    NN = N * N
    QN = Q * N
    r_i = jax.lax.broadcasted_iota(jnp.int32, (N, NN), 1)
    n_i = jax.lax.broadcasted_iota(jnp.int32, (N, NN), 0)
    S0 = (r_i // N == n_i).astype(f32)
    r2_i = jax.lax.broadcasted_iota(jnp.int32, (NN, N), 0)
    g_i = jax.lax.broadcasted_iota(jnp.int32, (NN, N), 1)
    TileG = (r2_i % N == g_i).astype(f32)
    R = _blockdiag_mask(Q, QN, 1, N)

    W_h = W_msg_ref[0:HID, :]
    W_e = W_msg_ref[HID:HID + EF, :]
    b_msg = b_msg_ref[...]
    W_u = W_u_ref[...]
    W_m = W_m_ref[...]
    b_u = b_u_ref[...]
    W_gh = W_g_ref[0:HID, :]
    W_gx = W_g_ref[HID:2 * HID, :]
    W_oh = W_o_ref[0:HID, :]
    W_ox = W_o_ref[HID:2 * HID, :]

    WkH = jnp.tile(W_h, (Q, Q)) * _blockdiag_mask(Q * HID, Q * MSG, HID, MSG)
    WkE = jnp.tile(W_e, (Q, Q)) * _blockdiag_mask(Q * EF, Q * MSG, EF, MSG)
    _r = jax.lax.broadcasted_iota(jnp.int32, (Q * Q * MSG, 1), 0)
    WmP = jnp.tile(W_m, (Q * Q, 1)) * (
        (_r // (Q * MSG)) == ((_r % (Q * MSG)) // MSG)).astype(f32)
    b_msg4 = jnp.tile(b_msg, (1, Q))
    A32 = jnp.tile(jnp.ones((EF, MSG), f32), (Q, Q)) * _blockdiag_mask(
        Q * EF, Q * MSG, EF, MSG)

    dot = functools.partial(jnp.dot, preferred_element_type=f32)

    for gb in range(GB):
        xs = [x_ref[gb * Q + q] for q in range(Q)]
        es = [e_ref[gb * Q + q] for q in range(Q)]
        x_cat = jnp.concatenate(xs, axis=0)

        E4 = b_msg4
        adj32 = jnp.zeros((NN, Q * MSG), f32)
        for q in range(Q):
            E4 = E4 + dot(es[q], WkE[q * EF:(q + 1) * EF, :])
            adj32 = adj32 + dot(es[q], A32[q * EF:(q + 1) * EF, :])
        mask4 = (adj32 != 0.0).astype(f32)

        acts = []
        for q in range(Q):
            adj_q = jnp.sum(es[q], axis=1, keepdims=True)
            acts.append(dot(S0, adj_q))
        act = jnp.concatenate(acts, axis=0) != 0.0

        h_cat = x_cat
        for _ in range(PASSES):
            Hm4 = jnp.zeros((N, Q * MSG), f32)
            for q in range(Q):
                Hm4 = Hm4 + dot(h_cat[q * N:(q + 1) * N, :],
                                WkH[q * HID:(q + 1) * HID, :])
            T4 = jnp.tanh(E4 + dot(TileG, Hm4))
            msg4 = dot(S0, T4 * mask4)
            mm = jnp.concatenate(
                [dot(msg4, WmP[q * Q * MSG:(q + 1) * Q * MSG, :])
                 for q in range(Q)], axis=0)
            upd = jnp.tanh(dot(h_cat, W_u) + mm + b_u)
            h_cat = jnp.where(act, upd, h_cat)
        gate = jax.nn.sigmoid(dot(h_cat, W_gh) + dot(x_cat, W_gx))
        o = dot(h_cat, W_oh) + dot(x_cat, W_ox)
        gated = gate * o * act.astype(f32)
        out_ref[gb] = dot(R, gated)


def kernel(nodes, edges, W_msg, b_msg, W_u, W_m, b_u, W_g, W_o):
    f32 = jnp.float32
    eflat = edges.reshape(B, N * N, EF)
    b_msg2 = b_msg.reshape(1, MSG)
    b_u2 = b_u.reshape(1, HID)

    full = lambda shape: pl.BlockSpec(shape, lambda i: (0,) * len(shape))
    out = pl.pallas_call(
        _mpnn_kernel,
        grid=(G // GB,),
        in_specs=[
            pl.BlockSpec((GB * Q, N, NF), lambda i: (i, 0, 0)),
            pl.BlockSpec((GB * Q, N * N, EF), lambda i: (i, 0, 0)),
            full((HID + EF, MSG)),
            full((1, MSG)),
            full((HID, HID)),
            full((MSG, HID)),
            full((1, HID)),
            full((2 * HID, HID)),
            full((2 * HID, HID)),
        ],
        out_specs=pl.BlockSpec((GB, Q, HID), lambda i: (i, 0, 0)),
        out_shape=jax.ShapeDtypeStruct((G, Q, HID), f32),
    )(nodes, eflat, W_msg, b_msg2, W_u, W_m, b_u2, W_g, W_o)
    return out.reshape(B, HID)
